# Initial kernel scaffold; baseline (speedup 1.0000x reference)
import jax
import jax.numpy as jnp
from jax.experimental import pallas as pl


def kernel(inputs):
    raise NotImplementedError("write your pallas kernel here")


# speedup vs baseline: 2.7484x; 2.7484x over previous
import jax
import jax.numpy as jnp
import numpy as np
from jax import lax
from jax.experimental import pallas as pl

_GAUSS = ((-1.645, 0.283), (-1.08, 0.17), (-0.739, 0.134), (-0.468, 0.118),
          (-0.228, 0.114), (0.0, 0.114), (0.228, 0.114), (0.468, 0.118),
          (0.739, 0.134), (1.08, 0.17), (1.645, 0.283))

_K = 11
_F = 128
_FK = _F * _K
_ROWS_PER_BLOCK = 256


def _dot(a, b):
    return lax.dot_general(a, b, (((1,), (0,)), ((), ())),
                           preferred_element_type=jnp.float32)


def _make_consts():
    f_of_o = np.arange(_FK) // _K
    e = np.zeros((_F, _FK), np.float32)
    e[f_of_o, np.arange(_FK)] = 1.0
    mus = np.array([m for m, _ in _GAUSS], np.float32)
    cs = np.array([-0.5 / (s * s) for _, s in _GAUSS], np.float32)
    muc = np.zeros((8, _FK), np.float32)
    muc[0] = mus[np.arange(_FK) % _K]
    muc[1] = cs[np.arange(_FK) % _K]
    return (jnp.asarray(e, jnp.bfloat16), jnp.asarray(e.T, jnp.bfloat16),
            jnp.asarray(muc))


def _weave_kernel(x_ref, e_ref, et_ref, muc_ref, o_ref):
    f32 = jnp.float32
    bf16 = jnp.bfloat16
    e_bf = e_ref[...]
    et_bf = et_ref[...]
    mu_vec = muc_ref[0:1, :]
    c_vec = muc_ref[1:2, :]

    x = x_ref[...]
    x_hi = x.astype(bf16)
    x_lo = (x - x_hi.astype(f32)).astype(bf16)
    xx = _dot(x_hi, e_bf) + _dot(x_lo, e_bf)

    d = xx - mu_vec
    p = jnp.exp(c_vec * (d * d))

    s = _dot(p.astype(bf16), et_bf)
    r = 1.0 / (s + 1e-9)
    rr = _dot(r.astype(bf16), e_bf)

    o_ref[...] = p * rr


def kernel(inputs):
    b, n, f = inputs.shape
    x = inputs.reshape(b * n, f)
    rows = b * n
    blk = _ROWS_PER_BLOCK
    e_bf, et_bf, muc = _make_consts()
    out = pl.pallas_call(
        _weave_kernel,
        grid=(rows // blk,),
        in_specs=[
            pl.BlockSpec((blk, f), lambda i: (i, 0)),
            pl.BlockSpec((_F, _FK), lambda i: (0, 0)),
            pl.BlockSpec((_FK, _F), lambda i: (0, 0)),
            pl.BlockSpec((8, _FK), lambda i: (0, 0)),
        ],
        out_specs=pl.BlockSpec((blk, f * _K), lambda i: (i, 0)),
        out_shape=jax.ShapeDtypeStruct((rows, f * _K), jnp.float32),
    )(x, e_bf, et_bf, muc)
    return out.reshape(b, n, f * _K)


# speedup vs baseline: 3.9900x; 1.4518x over previous
import jax
import jax.numpy as jnp
import numpy as np
from jax import lax
from jax.experimental import pallas as pl

_GAUSS = ((-1.645, 0.283), (-1.08, 0.17), (-0.739, 0.134), (-0.468, 0.118),
          (-0.228, 0.114), (0.0, 0.114), (0.228, 0.114), (0.468, 0.118),
          (0.739, 0.134), (1.08, 0.17), (1.645, 0.283))

_K = 11
_F = 128
_FK = _F * _K
_ROWS_PER_BLOCK = 512
_LN2 = float(np.log(2.0))


def _dot(a, b):
    return lax.dot_general(a, b, (((1,), (0,)), ((), ())),
                           preferred_element_type=jnp.float32)


def _make_consts():
    f_of_o = np.arange(_FK) // _K
    e = np.zeros((_F, _FK), np.float32)
    e[f_of_o, np.arange(_FK)] = 1.0
    e2 = np.concatenate([e, e], axis=0)
    mus = np.array([m for m, _ in _GAUSS], np.float32)
    cs = np.array([-0.5 / (s * s) for _, s in _GAUSS], np.float32)
    c2s = cs / np.float32(np.log(2.0))
    muc = np.zeros((8, _FK), np.float32)
    muc[0] = mus[np.arange(_FK) % _K]
    muc[1] = c2s[np.arange(_FK) % _K]
    return jnp.asarray(e2, jnp.bfloat16), jnp.asarray(muc)


def _hi_lo(v):
    hi = v.astype(jnp.bfloat16)
    lo = (v - hi.astype(jnp.float32)).astype(jnp.bfloat16)
    return jnp.concatenate([hi, lo], axis=1)


def _weave_kernel(x_ref, e2_ref, muc_ref, o_ref):
    f32 = jnp.float32
    e2_bf = e2_ref[...]
    mu_vec = muc_ref[0:1, :]
    c_vec = muc_ref[1:2, :]

    x = x_ref[...]
    s = None
    for mu, sig in _GAUSS:
        d = x - f32(mu)
        t = jnp.exp2(f32(-0.5 / (sig * sig * _LN2)) * (d * d))
        s = t if s is None else s + t
    ell = -jnp.log2(s + 1e-9)

    xx = _dot(_hi_lo(x), e2_bf)
    ll = _dot(_hi_lo(ell), e2_bf)

    d = xx - mu_vec
    o_ref[...] = jnp.exp2(c_vec * (d * d) + ll)


def kernel(inputs):
    b, n, f = inputs.shape
    x = inputs.reshape(b * n, f)
    rows = b * n
    blk = _ROWS_PER_BLOCK
    e2_bf, muc = _make_consts()
    out = pl.pallas_call(
        _weave_kernel,
        grid=(rows // blk,),
        in_specs=[
            pl.BlockSpec((blk, f), lambda i: (i, 0)),
            pl.BlockSpec((2 * _F, _FK), lambda i: (0, 0)),
            pl.BlockSpec((8, _FK), lambda i: (0, 0)),
        ],
        out_specs=pl.BlockSpec((blk, f * _K), lambda i: (i, 0)),
        out_shape=jax.ShapeDtypeStruct((rows, f * _K), jnp.float32),
    )(x, e2_bf, muc)
    return out.reshape(b, n, f * _K)


# speedup vs baseline: 4.7724x; 1.1961x over previous
import jax
import jax.numpy as jnp
import numpy as np
from jax import lax
from jax.experimental import pallas as pl

_GAUSS = ((-1.645, 0.283), (-1.08, 0.17), (-0.739, 0.134), (-0.468, 0.118),
          (-0.228, 0.114), (0.0, 0.114), (0.228, 0.114), (0.468, 0.118),
          (0.739, 0.134), (1.08, 0.17), (1.645, 0.283))

_K = 11
_F = 128
_FK = _F * _K
_ROWS_PER_BLOCK = 512
_LN2 = float(np.log(2.0))


def _dot(a, b):
    return lax.dot_general(a, b, (((1,), (0,)), ((), ())),
                           preferred_element_type=jnp.float32)


def _make_consts():
    f_of_o = np.arange(_FK) // _K
    e = np.zeros((_F, _FK), np.float32)
    e[f_of_o, np.arange(_FK)] = 1.0
    e2 = np.concatenate([e, e], axis=0)
    mus = np.array([m for m, _ in _GAUSS], np.float32)
    cs = np.array([-0.5 / (s * s) for _, s in _GAUSS], np.float32)
    c2s = cs / np.float32(_LN2)
    muc = np.zeros((8, _FK), np.float32)
    muc[0] = mus[np.arange(_FK) % _K]
    muc[1] = c2s[np.arange(_FK) % _K]
    return jnp.asarray(e2, jnp.bfloat16), jnp.asarray(muc)


def _hi_lo(v):
    hi = v.astype(jnp.bfloat16)
    lo = (v - hi.astype(jnp.float32)).astype(jnp.bfloat16)
    return jnp.concatenate([hi, lo], axis=1)


def _weave_kernel(x_ref, e2_ref, muc_ref, o_ref):
    f32 = jnp.float32
    e2_bf = e2_ref[...]
    mu_vec = muc_ref[0:1, :]
    c_vec = muc_ref[1:2, :]

    x = x_ref[...]
    x2 = x * x
    c2_mid = f32(-0.5 / (_GAUSS[5][1] ** 2 * _LN2))
    s = jnp.exp2(c2_mid * x2)
    for i in range(5):
        m, sig = _GAUSS[6 + i]
        c2 = f32(-0.5 / (sig * sig * _LN2))
        u = c2 * x2 + f32(c2 * m * m)
        v = f32(2.0 * c2 * m) * x
        s = s + jnp.exp2(u - v) + jnp.exp2(u + v)
    ell = -jnp.log2(s + 1e-9)

    xx = _dot(_hi_lo(x), e2_bf)

    lane = lax.broadcasted_iota(jnp.int32, (1, _F), 1)
    for j in range(_K):
        idx = (lane + 128 * j) // _K
        idxb = jnp.broadcast_to(idx, (x.shape[0], _F))
        ll = jnp.take_along_axis(ell, idxb, axis=1)
        c0 = 128 * j
        d = xx[:, c0:c0 + _F] - mu_vec[:, c0:c0 + _F]
        o_ref[:, c0:c0 + _F] = jnp.exp2(c_vec[:, c0:c0 + _F] * (d * d) + ll)


def kernel(inputs):
    b, n, f = inputs.shape
    x = inputs.reshape(b * n, f)
    rows = b * n
    blk = _ROWS_PER_BLOCK
    e2_bf, muc = _make_consts()
    out = pl.pallas_call(
        _weave_kernel,
        grid=(rows // blk,),
        in_specs=[
            pl.BlockSpec((blk, f), lambda i: (i, 0)),
            pl.BlockSpec((2 * _F, _FK), lambda i: (0, 0)),
            pl.BlockSpec((8, _FK), lambda i: (0, 0)),
        ],
        out_specs=pl.BlockSpec((blk, f * _K), lambda i: (i, 0)),
        out_shape=jax.ShapeDtypeStruct((rows, f * _K), jnp.float32),
    )(x, e2_bf, muc)
    return out.reshape(b, n, f * _K)
